# baseline (speedup 1.0000x reference)
import jax
import jax.numpy as jnp
from jax import lax
from jax.experimental import pallas as pl
from jax.experimental.pallas import tpu as pltpu
from jax.experimental.pallas import tpu_sc as plsc

B = 128
NQ = 900
NC = 80
NPI = NQ * NC
K = 300
ROWPAD = 304
ABUF = 352
NVREG = NPI // 16
NW = 32
IPW = B // NW


def _keys_body(x_ref, o_ref):
    o_ref[...] = lax.bitcast_convert_type(jax.nn.sigmoid(x_ref[...]), jnp.int32)


def _bsum(v, lanes):
    for sh in (1, 2, 4, 8):
        v = v + v.at[jnp.bitwise_xor(lanes, sh)].get(mode="promise_in_bounds")
    return v


def _bmax(v, lanes):
    for sh in (1, 2, 4, 8):
        v = jnp.maximum(v, v.at[jnp.bitwise_xor(lanes, sh)].get(mode="promise_in_bounds"))
    return v


def _bprefix_excl(t, lanes):
    v = t
    for sh in (1, 2, 4, 8):
        g = v.at[jnp.maximum(lanes - sh, 0)].get(mode="promise_in_bounds")
        v = v + jnp.where(lanes >= sh, g, 0)
    return v - t


def _scal(v, l=0):
    return lax.squeeze(lax.slice(v, (l,), (l + 1,)), (0,))


def _sc_select_body(keys_hbm, outk_hbm, outi_hbm, ebuf_hbm, meta_hbm,
                    kv, bak, bai, bei, rkb, sem):
    cid = lax.axis_index("c")
    sid = lax.axis_index("s")
    wid = sid * 2 + cid
    lanes = lax.iota(jnp.int32, 16)
    zeros16 = jnp.zeros((16,), jnp.int32)

    def per_image(t, _):
        img = wid * IPW + t
        base = img * NPI
        pltpu.sync_copy(keys_hbm.at[pl.ds(base, NPI)], kv)

        def count_ge(thr):
            def cb(i, acc):
                v = kv[pl.ds(i * 16, 16)]
                return acc + jnp.where(v >= thr, 1, 0)
            acc = lax.fori_loop(0, NVREG, cb, zeros16)
            return _scal(_bsum(acc, lanes))

        def bis(i, lh):
            lo, hi = lh
            mid = lo + lax.shift_right_logical(hi - lo, 1)
            c = count_ge(mid)
            big = c >= K
            lo2 = jnp.where(big, mid, lo)
            hi2 = jnp.where(big, hi, mid)
            return (lo2, hi2)

        lo, hi = lax.fori_loop(0, 30, bis, (jnp.int32(0), jnp.int32(1 << 30)))

        def sweep(i, offs):
            offa, offe = offs
            v = kv[pl.ds(i * 16, 16)]
            ma = v > lo
            me = v == lo
            ta = jnp.where(ma, 1, 0)
            te = jnp.where(me, 1, 0)
            packed = _bsum(ta * 256 + te, lanes)
            pk = _scal(packed)
            ca = lax.shift_right_logical(pk, 8)
            ce = jnp.bitwise_and(pk, 255)
            idxv = i * 16 + lanes

            @pl.when(ca == 1)
            def _():
                kb = _bmax(jnp.where(ma, v, 0), lanes)
                ib = _bmax(jnp.where(ma, idxv, -1), lanes)
                bak[pl.ds(offa, 16)] = kb
                bai[pl.ds(offa, 16)] = ib

            @pl.when(ca >= 2)
            def _():
                pa = _bprefix_excl(ta, lanes)
                for l in range(16):
                    tl = _scal(ta, l)
                    pal = _scal(pa, l)

                    @pl.when(tl == 1)
                    def _():
                        bak[pl.ds(offa + pal, 16)] = _scal(v, l) + zeros16
                        bai[pl.ds(offa + pal, 16)] = _scal(idxv, l) + zeros16

            @pl.when((ce >= 1) & (offe < 310))
            def _():
                pe = _bprefix_excl(te, lanes)
                for l in range(16):
                    tl = _scal(te, l)
                    pel = _scal(pe, l)

                    @pl.when(tl == 1)
                    def _():
                        bei[pl.ds(offe + pel, 16)] = _scal(idxv, l) + zeros16

            return (offa + ca, offe + ce)

        na, _ne = lax.fori_loop(0, NVREG, sweep, (jnp.int32(0), jnp.int32(0)))

        def rank_one(e, _):
            ve = bak[pl.ds(e, 16)]
            ie = bai[pl.ds(e, 16)]
            ke_b = _scal(ve) + zeros16
            ie_b = _scal(ie) + zeros16

            def rb(j, acc):
                kj = bak[pl.ds(j * 16, 16)]
                ij = bai[pl.ds(j * 16, 16)]
                valid = (j * 16 + lanes) < na
                beats = (kj > ke_b) | ((kj == ke_b) & (ij < ie_b))
                return acc + jnp.where(beats & valid, 1, 0)

            acc = lax.fori_loop(0, 20, rb, zeros16)
            rkb[pl.ds(e, 16)] = _bsum(acc, lanes) + img * 320
            return 0

        lax.fori_loop(0, na, rank_one, 0)

        padtgt = img * 320 + 300 + lanes
        for j in range(20):
            off_j = jnp.minimum(na + 16 * j, 304)
            rkb[pl.ds(off_j, 16)] = padtgt

        for j in range(20):
            tgt = rkb[pl.ds(16 * j, 16)]
            pltpu.async_copy(bak.at[pl.ds(16 * j, 16)], outk_hbm.at[tgt], sem).wait()
            pltpu.async_copy(bai.at[pl.ds(16 * j, 16)], outi_hbm.at[tgt], sem).wait()
        pltpu.sync_copy(bei.at[pl.ds(0, 320)], ebuf_hbm.at[pl.ds(img * 320, 320)])
        rkb[pl.ds(0, 16)] = jnp.where(lanes == 0, lo, na)
        pltpu.sync_copy(rkb.at[pl.ds(0, 16)], meta_hbm.at[pl.ds(img * 16, 16)])
        return 0

    lax.fori_loop(0, IPW, per_image, 0)


def _sc_emit_body(outk_hbm, outi_hbm, ebuf_hbm, meta_hbm, boxes_hbm, scale_hbm,
                  lab_hbm, sco_hbm, x_hbm, y_hbm, w_hbm, h_hbm,
                  sok, soi, bei, mv,
                  bi0, bi1, bi2, bi3, c0, c1, c2, c3,
                  pl_lab, pl_sco, pl_x, pl_y, pl_w, pl_h, scv, sem):
    cid = lax.axis_index("c")
    sid = lax.axis_index("s")
    wid = sid * 2 + cid
    lanes = lax.iota(jnp.int32, 16)
    zeros16 = jnp.zeros((16,), jnp.int32)

    pltpu.sync_copy(scale_hbm, scv)
    scvv = scv[...]
    w_sc = _scal(scvv, 0)
    h_sc = _scal(scvv, 1)

    def per_image(t, _):
        img = wid * IPW + t
        pltpu.sync_copy(outk_hbm.at[pl.ds(img * 320, 320)], sok)
        pltpu.sync_copy(outi_hbm.at[pl.ds(img * 320, 320)], soi)
        pltpu.sync_copy(ebuf_hbm.at[pl.ds(img * 320, 320)], bei)
        pltpu.sync_copy(meta_hbm.at[pl.ds(img * 16, 16)], mv)
        mvv = mv[...]
        lo = _scal(mvv, 0)
        na = _scal(mvv, 1)

        for j in range(19):
            src = bei[pl.ds(16 * j, 16)]
            off_j = jnp.minimum(na + 16 * j, 304)
            sok[pl.ds(off_j, 16)] = lo + zeros16
            soi[pl.ds(off_j, 16)] = src

        for j in range(19):
            kvj = sok[pl.ds(16 * j, 16)]
            ivj = soi[pl.ds(16 * j, 16)]
            ivj = jnp.clip(ivj, 0, NPI - 1)
            q = lax.shift_right_logical(
                lax.shift_right_logical(ivj, 4) * 13108, 16)
            labv = ivj - q * NC
            gbase = (img * NQ + q) * 4
            bi0[pl.ds(16 * j, 16)] = gbase
            bi1[pl.ds(16 * j, 16)] = gbase + 1
            bi2[pl.ds(16 * j, 16)] = gbase + 2
            bi3[pl.ds(16 * j, 16)] = gbase + 3
            pl_lab[pl.ds(16 * j, 16)] = labv.astype(jnp.float32)
            pl_sco[pl.ds(16 * j, 16)] = lax.bitcast_convert_type(kvj, jnp.float32)

        pltpu.async_copy(boxes_hbm.at[bi0], c0, sem).wait()
        pltpu.async_copy(boxes_hbm.at[bi1], c1, sem).wait()
        pltpu.async_copy(boxes_hbm.at[bi2], c2, sem).wait()
        pltpu.async_copy(boxes_hbm.at[bi3], c3, sem).wait()

        for j in range(19):
            cx = c0[pl.ds(16 * j, 16)]
            cy = c1[pl.ds(16 * j, 16)]
            bw = c2[pl.ds(16 * j, 16)]
            bh = c3[pl.ds(16 * j, 16)]
            pl_x[pl.ds(16 * j, 16)] = (cx - 0.5 * bw) * w_sc
            pl_y[pl.ds(16 * j, 16)] = (cy - 0.5 * bh) * h_sc
            pl_w[pl.ds(16 * j, 16)] = bw * w_sc
            pl_h[pl.ds(16 * j, 16)] = bh * h_sc

        ob = img * ROWPAD
        pltpu.sync_copy(pl_lab, lab_hbm.at[pl.ds(ob, ROWPAD)])
        pltpu.sync_copy(pl_sco, sco_hbm.at[pl.ds(ob, ROWPAD)])
        pltpu.sync_copy(pl_x, x_hbm.at[pl.ds(ob, ROWPAD)])
        pltpu.sync_copy(pl_y, y_hbm.at[pl.ds(ob, ROWPAD)])
        pltpu.sync_copy(pl_w, w_hbm.at[pl.ds(ob, ROWPAD)])
        pltpu.sync_copy(pl_h, h_hbm.at[pl.ds(ob, ROWPAD)])
        return 0

    lax.fori_loop(0, IPW, per_image, 0)


def kernel(logits, boxes, original_sizes):
    b, q, c = logits.shape
    n = b * q * c
    flat = logits.reshape(n // 1024, 1024)
    keys = pl.pallas_call(
        _keys_body,
        out_shape=jax.ShapeDtypeStruct(flat.shape, jnp.int32),
        grid=(9,),
        in_specs=[pl.BlockSpec((flat.shape[0] // 9, 1024), lambda i: (i, 0))],
        out_specs=pl.BlockSpec((flat.shape[0] // 9, 1024), lambda i: (i, 0)),
    )(flat)
    keys_flat = keys.reshape(n)

    os0 = original_sizes[0].astype(jnp.float32)
    wh = jnp.stack([os0[1], os0[0]])
    scale16 = jnp.tile(wh, 8)

    boxes_flat = boxes.reshape(-1)

    mesh = plsc.VectorSubcoreMesh(core_axis_name="c", subcore_axis_name="s")
    f32 = jnp.float32
    i32 = jnp.int32
    outk, outi, ebuf, meta = pl.kernel(
        _sc_select_body,
        out_type=(
            jax.ShapeDtypeStruct((B * 320,), i32),
            jax.ShapeDtypeStruct((B * 320,), i32),
            jax.ShapeDtypeStruct((B * 320,), i32),
            jax.ShapeDtypeStruct((B * 16,), i32),
        ),
        mesh=mesh,
        scratch_types=[
            pltpu.VMEM((NPI,), i32),
            pltpu.VMEM((320,), i32),
            pltpu.VMEM((320,), i32),
            pltpu.VMEM((ABUF,), i32),
            pltpu.VMEM((320,), i32),
            pltpu.SemaphoreType.DMA,
        ],
    )(keys_flat)

    outs = pl.kernel(
        _sc_emit_body,
        out_type=tuple(
            jax.ShapeDtypeStruct((B * ROWPAD,), f32) for _ in range(6)
        ),
        mesh=mesh,
        scratch_types=[
            pltpu.VMEM((320,), i32),
            pltpu.VMEM((320,), i32),
            pltpu.VMEM((320,), i32),
            pltpu.VMEM((16,), i32),
            pltpu.VMEM((ROWPAD,), i32),
            pltpu.VMEM((ROWPAD,), i32),
            pltpu.VMEM((ROWPAD,), i32),
            pltpu.VMEM((ROWPAD,), i32),
            pltpu.VMEM((ROWPAD,), f32),
            pltpu.VMEM((ROWPAD,), f32),
            pltpu.VMEM((ROWPAD,), f32),
            pltpu.VMEM((ROWPAD,), f32),
            pltpu.VMEM((ROWPAD,), f32),
            pltpu.VMEM((ROWPAD,), f32),
            pltpu.VMEM((ROWPAD,), f32),
            pltpu.VMEM((ROWPAD,), f32),
            pltpu.VMEM((ROWPAD,), f32),
            pltpu.VMEM((ROWPAD,), f32),
            pltpu.VMEM((16,), f32),
            pltpu.SemaphoreType.DMA,
        ],
    )(outk, outi, ebuf, meta, boxes_flat, scale16)

    lab, sco, x, y, w, h = outs
    planes = [p.reshape(B, ROWPAD)[:, :K] for p in (lab, sco, x, y, w, h)]
    return jnp.stack(planes, axis=-1)
